# baseline (speedup 1.0000x reference)
import functools

import jax
import jax.numpy as jnp
from jax import lax
from jax.experimental import pallas as pl
from jax.experimental.pallas import tpu as pltpu
from jax.experimental.pallas import tpu_sc as plsc

N_VOCAB = 2048
SEQ = 8192
SEQ_OUT = 8193
NC, NS = 2, 16
NW = NC * NS
W_PER_B = NW // 4
ROWS_PER_W = SEQ // W_PER_B
ZSLICE = 32768
ZWORDS = NS * ZSLICE
W_WORDS = ROWS_PER_W * N_VOCAB
N_ZDMA = W_WORDS // ZWORDS

_mesh = plsc.VectorSubcoreMesh(
    core_axis_name="c", subcore_axis_name="s", num_cores=NC, num_subcores=NS
)


@functools.partial(
    pl.kernel,
    out_type=jax.ShapeDtypeStruct((4 * SEQ_OUT * N_VOCAB,), jnp.float32),
    mesh=_mesh,
    scratch_types=[
        pltpu.VMEM((ROWS_PER_W,), jnp.int32),
        pltpu.VMEM((8, 128), jnp.int32),
        pltpu.VMEM((128,), jnp.float32),
        pltpu.VMEM((16,), jnp.float32),
        pltpu.VMEM((16,), jnp.int32),
        pltpu.VMEM((16,), jnp.int32),
        pltpu.VMEM_SHARED((ZWORDS,), jnp.float32),
        pltpu.SemaphoreType.DMA,
        pltpu.SemaphoreType.DMA,
    ],
    compiler_params=pltpu.CompilerParams(
        use_tc_tiling_on_sc=False, needs_layout_passes=False
    ),
)
def _sc_onehot(ids_hbm, zeros_hbm, tail_hbm, out_hbm,
               ids_v, idx_v, ones_v, ones16_v, tail_v, tidx_v, zeros_sp,
               zsem, ssem):
    cid = lax.axis_index("c")
    sid = lax.axis_index("s")
    wid = sid * NC + cid
    b = wid // W_PER_B
    row0 = b * SEQ_OUT + (wid % W_PER_B) * ROWS_PER_W
    word0 = row0 * N_VOCAB

    lane16 = jnp.arange(16, dtype=jnp.int32)

    pltpu.sync_copy(ids_hbm.at[pl.ds(wid * ROWS_PER_W, ROWS_PER_W)], ids_v)
    pltpu.sync_copy(zeros_hbm, zeros_sp.at[pl.ds(sid * ZSLICE, ZSLICE)])

    ones16_v[...] = jnp.full((16,), 1.0, jnp.float32)
    for c in range(8):
        ones_v[pl.ds(c * 16, 16)] = jnp.full((16,), 1.0, jnp.float32)
    for j in range(8):
        for c in range(8):
            n0 = j * 128 + c * 16
            ids_chunk = ids_v[pl.ds(n0, 16)]
            idx_v[j, pl.ds(c * 16, 16)] = (
                (row0 + n0 + lane16) * N_VOCAB + ids_chunk
            )

    plsc.subcore_barrier()

    for i in range(N_ZDMA):
        pltpu.async_copy(
            zeros_sp, out_hbm.at[pl.ds(word0 + i * ZWORDS, ZWORDS)], zsem
        )
    tail_words = (b * SEQ_OUT + SEQ) * N_VOCAB

    @pl.when(wid < 4)
    def _tail_zero():
        pltpu.async_copy(
            zeros_sp.at[pl.ds(0, N_VOCAB)],
            out_hbm.at[pl.ds((wid * SEQ_OUT + SEQ) * N_VOCAB, N_VOCAB)],
            zsem,
        )

    for i in range(N_ZDMA):
        pltpu.make_async_copy(
            zeros_sp, out_hbm.at[pl.ds(word0, ZWORDS)], zsem
        ).wait()

    @pl.when(wid < 4)
    def _tail_zero_wait():
        pltpu.make_async_copy(
            zeros_sp.at[pl.ds(0, N_VOCAB)],
            out_hbm.at[pl.ds(tail_words, N_VOCAB)],
            zsem,
        ).wait()

    for j in range(8):
        pltpu.async_copy(ones_v, out_hbm.at[idx_v.at[j]], ssem)
    for j in range(8):
        pltpu.make_async_copy(ones_v, out_hbm.at[idx_v.at[j]], ssem).wait()

    @pl.when(wid < 4)
    def _tail_one():
        pltpu.sync_copy(tail_hbm, tail_v)
        last_id = plsc.load_gather(tail_v, [jnp.full((16,), wid, jnp.int32)])
        tidx_v[...] = (wid * SEQ_OUT + SEQ) * N_VOCAB + last_id
        pltpu.sync_copy(ones16_v, out_hbm.at[tidx_v])


def kernel(input_ids):
    ids = input_ids.astype(jnp.int32)
    shifted = jnp.pad(ids, ((0, 0), (1, 0)))[:, :SEQ].reshape(-1)
    tail = jnp.pad(ids[:, -1], (0, 12))
    zeros_blk = jnp.zeros((ZSLICE,), jnp.float32)
    out_flat = _sc_onehot(shifted, zeros_blk, tail)
    return out_flat.reshape(4, SEQ_OUT, N_VOCAB)


# speedup vs baseline: 1.0225x; 1.0225x over previous
import functools

import jax
import jax.numpy as jnp
from jax import lax
from jax.experimental import pallas as pl
from jax.experimental.pallas import tpu as pltpu
from jax.experimental.pallas import tpu_sc as plsc

N_VOCAB = 2048
SEQ = 8192
SEQ_OUT = 8193
N_ROWS = 4 * SEQ_OUT
NC, NS = 2, 16
NW = NC * NS
W_PER_B = NW // 4
ROWS_PER_W = SEQ // W_PER_B
G = 16
NGROUPS = ROWS_PER_W // G
K_RING = 3
NMAIN = (NGROUPS // K_RING) * K_RING

_mesh = plsc.VectorSubcoreMesh(
    core_axis_name="c", subcore_axis_name="s", num_cores=NC, num_subcores=NS
)


@functools.partial(
    pl.kernel,
    out_type=jax.ShapeDtypeStruct((N_ROWS, N_VOCAB), jnp.float32),
    mesh=_mesh,
    scratch_types=[
        pltpu.VMEM((ROWS_PER_W,), jnp.int32),
        [pltpu.VMEM((G, N_VOCAB), jnp.float32)] * K_RING,
        [pltpu.VMEM((G,), jnp.int32)] * K_RING,
        pltpu.VMEM((16,), jnp.int32),
        [pltpu.SemaphoreType.DMA] * K_RING,
    ],
    compiler_params=pltpu.CompilerParams(
        use_tc_tiling_on_sc=False, needs_layout_passes=False
    ),
)
def _sc_onehot(ids_hbm, zeros_hbm, tail_hbm, out_hbm,
               ids_v, bufs, ridxs, tail_v, sems):
    cid = lax.axis_index("c")
    sid = lax.axis_index("s")
    wid = sid * NC + cid
    b = wid // W_PER_B
    row0 = b * SEQ_OUT + (wid % W_PER_B) * ROWS_PER_W

    rows16 = jnp.arange(G, dtype=jnp.int32)
    ones = jnp.full((G,), 1.0, jnp.float32)
    zeros16 = jnp.zeros((G,), jnp.float32)

    pltpu.sync_copy(ids_hbm.at[pl.ds(wid * ROWS_PER_W, ROWS_PER_W)], ids_v)
    for k in range(K_RING):
        pltpu.sync_copy(zeros_hbm, bufs[k])

    def put(buf, g, vals):
        idx = ids_v[pl.ds(g * G, G)]
        plsc.store_scatter(buf, [rows16, idx], vals)

    def start(k, g):
        ridxs[k][...] = row0 + g * G + rows16
        pltpu.async_copy(bufs[k], out_hbm.at[ridxs[k]], sems[k])

    def drain(k):
        pltpu.make_async_copy(bufs[k], out_hbm.at[ridxs[k]], sems[k]).wait()

    for g in range(K_RING):
        put(bufs[g], g, ones)
        start(g, g)

    def body(h, carry):
        for k in range(K_RING):
            g = K_RING * h + k
            drain(k)
            put(bufs[k], g - K_RING, zeros16)
            put(bufs[k], g, ones)
            start(k, g)
        return carry

    lax.fori_loop(1, NGROUPS // K_RING, body, 0)

    for g in range(NMAIN, NGROUPS):
        k = g % K_RING
        drain(k)
        put(bufs[k], g - K_RING, zeros16)
        put(bufs[k], g, ones)
        start(k, g)

    for k in range(K_RING):
        drain(k)

    k_last = (NGROUPS - 1) % K_RING

    @pl.when(wid < 4)
    def _tail():
        put(bufs[k_last], NGROUPS - 1, zeros16)
        pltpu.sync_copy(tail_hbm, tail_v)
        last_id = plsc.load_gather(tail_v, [jnp.full((G,), wid, jnp.int32)])
        plsc.store_scatter(bufs[k_last], [rows16, last_id], ones,
                           mask=rows16 == 0)
        pltpu.sync_copy(bufs[k_last].at[pl.ds(0, 1), :],
                        out_hbm.at[pl.ds(wid * SEQ_OUT + SEQ, 1), :])


def kernel(input_ids):
    ids = input_ids.astype(jnp.int32)
    shifted = jnp.pad(ids, ((0, 0), (1, 0)))[:, :SEQ].reshape(-1)
    tail = jnp.pad(ids[:, -1], (0, 12))
    zeros_blk = jnp.zeros((G, N_VOCAB), jnp.float32)
    out_flat = _sc_onehot(shifted, zeros_blk, tail)
    return out_flat.reshape(4, SEQ_OUT, N_VOCAB)


# speedup vs baseline: 19.5567x; 19.1272x over previous
import jax
import jax.numpy as jnp
from jax import lax
from jax.experimental import pallas as pl

N_VOCAB = 2048
BATCH = 4
SEQ = 8192
SEQ_OUT = 8193
POS_BLK = 512
N_BLKS = 17


def _onehot_block(ids_ref, out_ref):
    ids = ids_ref[...]
    cls = lax.broadcasted_iota(jnp.int32, (POS_BLK, BATCH, N_VOCAB), 2)
    out_ref[...] = (ids[:, :, None] == cls).astype(jnp.float32)


def kernel(input_ids):
    ids_t = jnp.pad(
        input_ids.astype(jnp.int32).T,
        ((1, N_BLKS * POS_BLK - SEQ - 1), (0, 0)),
        constant_values=0,
    )

    out2 = pl.pallas_call(
        _onehot_block,
        grid=(N_BLKS,),
        in_specs=[pl.BlockSpec((POS_BLK, BATCH), lambda j: (j, 0))],
        out_specs=pl.BlockSpec((POS_BLK, BATCH, N_VOCAB), lambda j: (j, 0, 0)),
        out_shape=jax.ShapeDtypeStruct((SEQ_OUT, BATCH, N_VOCAB), jnp.float32),
    )(ids_t)
    return jnp.transpose(out2, (1, 0, 2))


# speedup vs baseline: 20.4568x; 1.0460x over previous
import jax
import jax.numpy as jnp
from jax import lax
from jax.experimental import pallas as pl

N_VOCAB = 2048
BATCH = 4
SEQ = 8192
SEQ_OUT = 8193
POS_BLK = 512
N_BLKS = 17


def _onehot_block(ids_ref, out_ref):
    ids = ids_ref[...].T
    cls = lax.broadcasted_iota(jnp.int32, (POS_BLK, BATCH, N_VOCAB), 2)
    out_ref[...] = (ids[:, :, None] == cls).astype(jnp.float32)


def kernel(input_ids):
    padded = jnp.pad(
        input_ids.astype(jnp.int32),
        ((0, 0), (1, N_BLKS * POS_BLK - SEQ - 1)),
        constant_values=0,
    )

    out2 = pl.pallas_call(
        _onehot_block,
        grid=(N_BLKS,),
        in_specs=[pl.BlockSpec((BATCH, POS_BLK), lambda j: (0, j))],
        out_specs=pl.BlockSpec((POS_BLK, BATCH, N_VOCAB), lambda j: (j, 0, 0)),
        out_shape=jax.ShapeDtypeStruct((SEQ_OUT, BATCH, N_VOCAB), jnp.float32),
    )(padded)
    return jnp.transpose(out2, (1, 0, 2))


# speedup vs baseline: 20.7833x; 1.0160x over previous
import jax
import jax.numpy as jnp
from jax import lax
from jax.experimental import pallas as pl

N_VOCAB = 2048
BATCH = 4
SEQ = 8192
SEQ_OUT = 8193
POS_BLK = 512
N_BLKS = 17


def _onehot_block(prev_ref, cur_ref, out_ref):
    j = pl.program_id(0)
    prev_last = prev_ref[:, POS_BLK - 1:]
    cur_head = cur_ref[:, : POS_BLK - 1]
    ids = jnp.concatenate([prev_last, cur_head], axis=1)
    lane = lax.broadcasted_iota(jnp.int32, (BATCH, POS_BLK), 1)
    ids = jnp.where((j == 0) & (lane == 0), 0, ids)
    cls = lax.broadcasted_iota(jnp.int32, (POS_BLK, BATCH, N_VOCAB), 2)
    out_ref[...] = (ids.T[:, :, None] == cls).astype(jnp.float32)


def kernel(input_ids):
    ids = input_ids.astype(jnp.int32)
    out2 = pl.pallas_call(
        _onehot_block,
        grid=(N_BLKS,),
        in_specs=[
            pl.BlockSpec((BATCH, POS_BLK), lambda j: (0, jnp.maximum(j - 1, 0))),
            pl.BlockSpec((BATCH, POS_BLK), lambda j: (0, jnp.minimum(j, SEQ // POS_BLK - 1))),
        ],
        out_specs=pl.BlockSpec((POS_BLK, BATCH, N_VOCAB), lambda j: (j, 0, 0)),
        out_shape=jax.ShapeDtypeStruct((SEQ_OUT, BATCH, N_VOCAB), jnp.float32),
    )(ids, ids)
    return jnp.transpose(out2, (1, 0, 2))


# speedup vs baseline: 20.8203x; 1.0018x over previous
import jax
import jax.numpy as jnp
from jax import lax
from jax.experimental import pallas as pl

N_VOCAB = 2048
BATCH = 4
SEQ = 8192
SEQ_OUT = 8193
POS_BLK = 512
N_BLKS = 17


def _onehot_block(prev_ref, cur_ref, out_ref):
    j = pl.program_id(0)

    @pl.when(j < N_BLKS - 1)
    def _full():
        prev_last = prev_ref[:, POS_BLK - 1:]
        cur_head = cur_ref[:, : POS_BLK - 1]
        ids = jnp.concatenate([prev_last, cur_head], axis=1)
        lane = lax.broadcasted_iota(jnp.int32, (BATCH, POS_BLK), 1)
        ids = jnp.where((j == 0) & (lane == 0), 0, ids)
        cls = lax.broadcasted_iota(jnp.int32, (POS_BLK, BATCH, N_VOCAB), 2)
        out_ref[...] = (ids.T[:, :, None] == cls).astype(jnp.float32)

    @pl.when(j == N_BLKS - 1)
    def _tail():
        prev_last = prev_ref[:, POS_BLK - 1:]
        cur_head = cur_ref[:, :7]
        ids8 = jnp.concatenate([prev_last, cur_head], axis=1)
        cls8 = lax.broadcasted_iota(jnp.int32, (8, BATCH, N_VOCAB), 2)
        out_ref[pl.ds(0, 8)] = (ids8.T[:, :, None] == cls8).astype(jnp.float32)


def kernel(input_ids):
    ids = input_ids.astype(jnp.int32)
    out2 = pl.pallas_call(
        _onehot_block,
        grid=(N_BLKS,),
        in_specs=[
            pl.BlockSpec((BATCH, POS_BLK), lambda j: (0, jnp.maximum(j - 1, 0))),
            pl.BlockSpec((BATCH, POS_BLK), lambda j: (0, jnp.minimum(j, SEQ // POS_BLK - 1))),
        ],
        out_specs=pl.BlockSpec((POS_BLK, BATCH, N_VOCAB), lambda j: (j, 0, 0)),
        out_shape=jax.ShapeDtypeStruct((SEQ_OUT, BATCH, N_VOCAB), jnp.float32),
    )(ids, ids)
    return jnp.transpose(out2, (1, 0, 2))


# speedup vs baseline: 21.3707x; 1.0264x over previous
import jax
import jax.numpy as jnp
from jax import lax
from jax.experimental import pallas as pl

N_VOCAB = 2048
BATCH = 4
SEQ = 8192
SEQ_OUT = 8193
POS_BLK = 256
N_BLKS = 33


def _onehot_block(prev_ref, cur_ref, out_ref):
    j = pl.program_id(0)

    @pl.when(j < N_BLKS - 1)
    def _full():
        prev_last = prev_ref[:, POS_BLK - 1:]
        cur_head = cur_ref[:, : POS_BLK - 1]
        ids = jnp.concatenate([prev_last, cur_head], axis=1)
        lane = lax.broadcasted_iota(jnp.int32, (BATCH, POS_BLK), 1)
        ids = jnp.where((j == 0) & (lane == 0), 0, ids)
        cls = lax.broadcasted_iota(jnp.int32, (POS_BLK, BATCH, N_VOCAB), 2)
        out_ref[...] = (ids.T[:, :, None] == cls).astype(jnp.float32)

    @pl.when(j == N_BLKS - 1)
    def _tail():
        prev_last = prev_ref[:, POS_BLK - 1:]
        cur_head = cur_ref[:, :7]
        ids8 = jnp.concatenate([prev_last, cur_head], axis=1)
        cls8 = lax.broadcasted_iota(jnp.int32, (8, BATCH, N_VOCAB), 2)
        out_ref[pl.ds(0, 8)] = (ids8.T[:, :, None] == cls8).astype(jnp.float32)


def kernel(input_ids):
    ids = input_ids.astype(jnp.int32)
    out2 = pl.pallas_call(
        _onehot_block,
        grid=(N_BLKS,),
        in_specs=[
            pl.BlockSpec((BATCH, POS_BLK), lambda j: (0, jnp.maximum(j - 1, 0))),
            pl.BlockSpec((BATCH, POS_BLK), lambda j: (0, jnp.minimum(j, SEQ // POS_BLK - 1))),
        ],
        out_specs=pl.BlockSpec((POS_BLK, BATCH, N_VOCAB), lambda j: (j, 0, 0)),
        out_shape=jax.ShapeDtypeStruct((SEQ_OUT, BATCH, N_VOCAB), jnp.float32),
    )(ids, ids)
    return jnp.transpose(out2, (1, 0, 2))
